# Initial kernel scaffold; baseline (speedup 1.0000x reference)
import jax
import jax.numpy as jnp
from jax.experimental import pallas as pl


def kernel(x, src_mask, max_dur, params):
    raise NotImplementedError("write your pallas kernel here")


# speedup vs baseline: 4.7559x; 4.7559x over previous
import functools

import jax
import jax.numpy as jnp
from jax import lax
from jax.experimental import pallas as pl
from jax.experimental.pallas import tpu as pltpu
from jax.experimental.pallas import tpu_sc as plsc

_B, _T, _D, _NBINS, _MAXD = 16, 512, 256, 256, 1024
_HI = jax.lax.Precision.HIGHEST
_DEF = jax.lax.Precision.DEFAULT


def _conv3(xb, wk0, wk1, wk2, bias):
    z = jnp.zeros((1, xb.shape[1]), jnp.float32)
    xm1 = jnp.concatenate([z, xb[:-1]], axis=0)
    xp1 = jnp.concatenate([xb[1:], z], axis=0)
    acc = jnp.dot(xm1, wk0, precision=_DEF)
    acc = acc + jnp.dot(xb, wk1, precision=_DEF)
    acc = acc + jnp.dot(xp1, wk2, precision=_DEF)
    return acc + bias


def _ln(h, g, b):
    mu = jnp.mean(h, axis=-1, keepdims=True)
    var = jnp.mean((h - mu) ** 2, axis=-1, keepdims=True)
    return (h - mu) / jnp.sqrt(var + 1e-5) * g + b


def _pred_chain(xb, W):
    (w10, w11, w12, b1, g1, bb1, w20, w21, w22, b2, g2, bb2, lw, lb) = W
    h = jnp.maximum(_conv3(xb, w10, w11, w12, b1), 0.0)
    h = _ln(h, g1, bb1)
    h = jnp.maximum(_conv3(h, w20, w21, w22, b2), 0.0)
    h = _ln(h, g2, bb2)
    return jnp.dot(h, lw, precision=_DEF) + lb


def _load_w(refs):
    return tuple(r[...] for r in refs)


def _tc1_body(x_ref, m_ref, md_ref,
              w10, w11, w12, b1, g1, bb1, w20, w21, w22, b2, g2, bb2, lw, lb,
              ld_ref, dv_ref, idx_ref, mlen_ref):
    b = pl.program_id(0)
    xb = x_ref[0]
    W = _load_w((w10, w11, w12, b1, g1, bb1, w20, w21, w22, b2, g2, bb2, lw, lb))
    ld = _pred_chain(xb, W)
    ld = jnp.where(m_ref[0] != 0, 0.0, ld)
    ld_ref[0] = ld
    dv = jnp.clip(jnp.round(jnp.exp(ld) - 1.0), 0.0, None)
    dv_ref[0] = dv
    ii = lax.broadcasted_iota(jnp.int32, (_T, _T), 0)
    jj = lax.broadcasted_iota(jnp.int32, (_T, _T), 1)
    tril = (jj <= ii).astype(jnp.float32)
    csum = jnp.dot(tril, dv, precision=_HI)
    mlen = jnp.minimum(csum[_T - 1:_T, :], md_ref[...])
    mlen_ref[0] = mlen
    pos = lax.broadcasted_iota(jnp.int32, (1, _MAXD), 1).astype(jnp.float32)
    cnt = jnp.sum((csum <= pos).astype(jnp.float32), axis=0, keepdims=True)
    valid = pos < mlen
    idx_ref[0] = jnp.where(valid, cnt, 0.0).astype(jnp.int32) + b * _T


def _run_tc1(x, maskf, md, W):
    wspec = [pl.BlockSpec(w.shape, lambda b: (0,) * w.ndim) for w in W]
    return pl.pallas_call(
        _tc1_body,
        grid=(_B,),
        in_specs=[pl.BlockSpec((1, _T, _D), lambda b: (b, 0, 0)),
                  pl.BlockSpec((1, _T, 1), lambda b: (b, 0, 0)),
                  pl.BlockSpec((1, 1), lambda b: (0, 0))] + wspec,
        out_specs=[pl.BlockSpec((1, _T, 1), lambda b: (b, 0, 0)),
                   pl.BlockSpec((1, _T, 1), lambda b: (b, 0, 0)),
                   pl.BlockSpec((1, 1, _MAXD), lambda b: (b, 0, 0)),
                   pl.BlockSpec((1, 1, 1), lambda b: (b, 0, 0))],
        out_shape=[jax.ShapeDtypeStruct((_B, _T, 1), jnp.float32),
                   jax.ShapeDtypeStruct((_B, _T, 1), jnp.float32),
                   jax.ShapeDtypeStruct((_B, 1, _MAXD), jnp.int32),
                   jax.ShapeDtypeStruct((_B, 1, 1), jnp.float32)],
    )(x, maskf, md, *W)


def _tc2_body(xr_ref, mlen_ref,
              pw10, pw11, pw12, pb1, pg1, pbb1, pw20, pw21, pw22, pb2, pg2, pbb2, plw, plb,
              ew10, ew11, ew12, eb1, eg1, ebb1, ew20, ew21, ew22, eb2, eg2, ebb2, elw, elb,
              pbins_ref, ebins_ref,
              xr3_ref, mm_ref, pp_ref, ep_ref, pidx_ref, eidx_ref):
    pos = lax.broadcasted_iota(jnp.int32, (_MAXD, 1), 0).astype(jnp.float32)
    mlen = mlen_ref[0]
    maskc = pos >= mlen
    mm_ref[0] = maskc.astype(jnp.int32)
    xb = jnp.where(maskc, 0.0, xr_ref[0])
    PW = _load_w((pw10, pw11, pw12, pb1, pg1, pbb1, pw20, pw21, pw22, pb2, pg2, pbb2, plw, plb))
    pp = _pred_chain(xb, PW)
    pp = jnp.where(maskc, 0.0, pp)
    pp_ref[0] = pp
    pidx = jnp.sum((pbins_ref[...] < pp).astype(jnp.int32), axis=-1, keepdims=True)
    pidx_ref[0] = pidx
    xr2 = xb + pp
    EW = _load_w((ew10, ew11, ew12, eb1, eg1, ebb1, ew20, ew21, ew22, eb2, eg2, ebb2, elw, elb))
    ep = _pred_chain(xr2, EW)
    ep = jnp.where(maskc, 0.0, ep)
    ep_ref[0] = ep
    eidx = jnp.sum((ebins_ref[...] < ep).astype(jnp.int32), axis=-1, keepdims=True)
    eidx_ref[0] = eidx + _NBINS
    xr3_ref[0] = xr2 + ep


def _run_tc2(xr, mlen, PW, EW, pbins, ebins):
    wspec = [pl.BlockSpec(w.shape, lambda b: (0,) * w.ndim) for w in PW + EW]
    return pl.pallas_call(
        _tc2_body,
        grid=(_B,),
        in_specs=[pl.BlockSpec((1, _MAXD, _D), lambda b: (b, 0, 0)),
                  pl.BlockSpec((1, 1, 1), lambda b: (b, 0, 0))] + wspec +
                 [pl.BlockSpec((1, _NBINS), lambda b: (0, 0)),
                  pl.BlockSpec((1, _NBINS), lambda b: (0, 0))],
        out_specs=[pl.BlockSpec((1, _MAXD, _D), lambda b: (b, 0, 0)),
                   pl.BlockSpec((1, _MAXD, 1), lambda b: (b, 0, 0)),
                   pl.BlockSpec((1, _MAXD, 1), lambda b: (b, 0, 0)),
                   pl.BlockSpec((1, _MAXD, 1), lambda b: (b, 0, 0)),
                   pl.BlockSpec((1, _MAXD, 1), lambda b: (b, 0, 0)),
                   pl.BlockSpec((1, _MAXD, 1), lambda b: (b, 0, 0))],
        out_shape=[jax.ShapeDtypeStruct((_B, _MAXD, _D), jnp.float32),
                   jax.ShapeDtypeStruct((_B, _MAXD, 1), jnp.int32),
                   jax.ShapeDtypeStruct((_B, _MAXD, 1), jnp.float32),
                   jax.ShapeDtypeStruct((_B, _MAXD, 1), jnp.float32),
                   jax.ShapeDtypeStruct((_B, _MAXD, 1), jnp.int32),
                   jax.ShapeDtypeStruct((_B, _MAXD, 1), jnp.int32)],
    )(xr, mlen, *PW, *EW, pbins, ebins)


def _sc_gather(table, idx):
    n, d = idx.shape[0], table.shape[1]
    nw = 32
    per_w = n // nw
    ch = 128
    n_ch = per_w // ch
    mesh = plsc.VectorSubcoreMesh(core_axis_name="c", subcore_axis_name="s")

    @functools.partial(
        pl.kernel, mesh=mesh,
        out_type=jax.ShapeDtypeStruct((n, d), jnp.float32),
        scratch_types=[pltpu.VMEM((per_w,), jnp.int32),
                       pltpu.VMEM((ch, d), jnp.float32),
                       pltpu.VMEM((ch, d), jnp.float32),
                       pltpu.SemaphoreType.DMA,
                       pltpu.SemaphoreType.DMA],
    )
    def k(table_hbm, idx_hbm, out_hbm, idx_v, buf0, buf1, sem0, sem1):
        wid = lax.axis_index("s") * 2 + lax.axis_index("c")
        base = wid * per_w
        pltpu.sync_copy(idx_hbm.at[pl.ds(base, per_w)], idx_v)
        bufs, sems = (buf0, buf1), (sem0, sem1)
        handles = [None] * n_ch
        handles[0] = pltpu.async_copy(
            table_hbm.at[idx_v.at[pl.ds(0, ch)]], bufs[0], sems[0])
        for ci in range(n_ch):
            if ci + 1 < n_ch:
                handles[ci + 1] = pltpu.async_copy(
                    table_hbm.at[idx_v.at[pl.ds((ci + 1) * ch, ch)]],
                    bufs[(ci + 1) % 2], sems[(ci + 1) % 2])
            handles[ci].wait()
            pltpu.sync_copy(bufs[ci % 2], out_hbm.at[pl.ds(base + ci * ch, ch)])

    return k(table, idx)


def _wpack(p):
    w1, w2 = p["conv1_w"], p["conv2_w"]
    return (jnp.asarray(w1[:, :, 0].T), jnp.asarray(w1[:, :, 1].T), jnp.asarray(w1[:, :, 2].T),
            p["conv1_b"].reshape(1, _D), p["ln1_g"].reshape(1, _D), p["ln1_b"].reshape(1, _D),
            jnp.asarray(w2[:, :, 0].T), jnp.asarray(w2[:, :, 1].T), jnp.asarray(w2[:, :, 2].T),
            p["conv2_b"].reshape(1, _D), p["ln2_g"].reshape(1, _D), p["ln2_b"].reshape(1, _D),
            p["lin_w"].reshape(_D, 1), p["lin_b"].reshape(1, 1))


def kernel(x, src_mask, max_dur, params):
    maskf = src_mask.astype(jnp.float32).reshape(_B, _T, 1)
    md = jnp.asarray(max_dur, jnp.float32).reshape(1, 1)
    DW = _wpack(params["dur"])
    PW = _wpack(params["pitch"])
    EW = _wpack(params["energy"])
    inf = jnp.full((1,), jnp.inf, jnp.float32)
    pbins = jnp.concatenate([params["pitch_bins"], inf]).reshape(1, _NBINS)
    ebins = jnp.concatenate([params["energy_bins"], inf]).reshape(1, _NBINS)

    ld3, dv3, idxg, mlen = _run_tc1(x, maskf, md, DW)
    xr = _sc_gather(x.reshape(_B * _T, _D), idxg.reshape(-1))
    xr = xr.reshape(_B, _MAXD, _D)
    xr3, mm, pp3, ep3, pidx3, eidx3 = _run_tc2(xr, mlen, PW, EW, pbins, ebins)
    emb_tab = jnp.concatenate([params["pitch_emb"], params["energy_emb"]], axis=0)
    allidx = jnp.concatenate([pidx3.reshape(-1), eidx3.reshape(-1)], axis=0)
    embs = _sc_gather(emb_tab, allidx)
    pitch_emb = embs[:_B * _MAXD].reshape(_B, _MAXD, _D)
    energy_emb = embs[_B * _MAXD:].reshape(_B, _MAXD, _D)
    return (xr3, mm.reshape(_B, _MAXD).astype(bool),
            ld3.reshape(_B, _T), dv3.reshape(_B, _T),
            pp3.reshape(_B, _MAXD), pitch_emb,
            ep3.reshape(_B, _MAXD), energy_emb)


# speedup vs baseline: 17.4805x; 3.6755x over previous
import functools

import jax
import jax.numpy as jnp
from jax import lax
from jax.experimental import pallas as pl
from jax.experimental.pallas import tpu as pltpu
from jax.experimental.pallas import tpu_sc as plsc

_B, _T, _D, _NBINS, _MAXD = 16, 512, 256, 256, 1024
_HI = jax.lax.Precision.HIGHEST
_DEF = jax.lax.Precision.DEFAULT


def _conv3(xb, wk0, wk1, wk2, bias):
    z = jnp.zeros((1, xb.shape[1]), jnp.float32)
    xm1 = jnp.concatenate([z, xb[:-1]], axis=0)
    xp1 = jnp.concatenate([xb[1:], z], axis=0)
    acc = jnp.dot(xm1, wk0, precision=_DEF)
    acc = acc + jnp.dot(xb, wk1, precision=_DEF)
    acc = acc + jnp.dot(xp1, wk2, precision=_DEF)
    return acc + bias


def _ln(h, g, b):
    mu = jnp.mean(h, axis=-1, keepdims=True)
    var = jnp.mean((h - mu) ** 2, axis=-1, keepdims=True)
    return (h - mu) / jnp.sqrt(var + 1e-5) * g + b


def _pred_chain(xb, W):
    (w10, w11, w12, b1, g1, bb1, w20, w21, w22, b2, g2, bb2, lw, lb) = W
    h = jnp.maximum(_conv3(xb, w10, w11, w12, b1), 0.0)
    h = _ln(h, g1, bb1)
    h = jnp.maximum(_conv3(h, w20, w21, w22, b2), 0.0)
    h = _ln(h, g2, bb2)
    return jnp.dot(h, lw, precision=_DEF) + lb


def _load_w(refs):
    return tuple(r[...] for r in refs)


def _tc1_body(x_ref, m_ref, md_ref,
              w10, w11, w12, b1, g1, bb1, w20, w21, w22, b2, g2, bb2, lw, lb,
              ld_ref, dv_ref, idx_ref, mlen_ref):
    b = pl.program_id(0)
    xb = x_ref[0]
    W = _load_w((w10, w11, w12, b1, g1, bb1, w20, w21, w22, b2, g2, bb2, lw, lb))
    ld = _pred_chain(xb, W)
    ld = jnp.where(m_ref[0] != 0, 0.0, ld)
    ld_ref[0] = ld
    dv = jnp.clip(jnp.round(jnp.exp(ld) - 1.0), 0.0, None)
    dv_ref[0] = dv
    ii = lax.broadcasted_iota(jnp.int32, (_T, _T), 0)
    jj = lax.broadcasted_iota(jnp.int32, (_T, _T), 1)
    tril = (jj <= ii).astype(jnp.float32)
    csum = jnp.dot(tril, dv, precision=_HI)
    mlen = jnp.minimum(csum[_T - 1:_T, :], md_ref[...])
    mlen_ref[0] = mlen
    pos = lax.broadcasted_iota(jnp.int32, (1, _MAXD), 1).astype(jnp.float32)
    cnt = jnp.sum((csum <= pos).astype(jnp.float32), axis=0, keepdims=True)
    valid = pos < mlen
    idx_ref[0] = jnp.where(valid, cnt, 0.0).astype(jnp.int32) + b * _T


def _run_tc1(x, maskf, md, W):
    wspec = [pl.BlockSpec(w.shape, lambda b: (0,) * w.ndim) for w in W]
    return pl.pallas_call(
        _tc1_body,
        grid=(_B,),
        in_specs=[pl.BlockSpec((1, _T, _D), lambda b: (b, 0, 0)),
                  pl.BlockSpec((1, _T, 1), lambda b: (b, 0, 0)),
                  pl.BlockSpec((1, 1), lambda b: (0, 0))] + wspec,
        out_specs=[pl.BlockSpec((1, _T, 1), lambda b: (b, 0, 0)),
                   pl.BlockSpec((1, _T, 1), lambda b: (b, 0, 0)),
                   pl.BlockSpec((1, 1, _MAXD), lambda b: (b, 0, 0)),
                   pl.BlockSpec((1, 1, 1), lambda b: (b, 0, 0))],
        out_shape=[jax.ShapeDtypeStruct((_B, _T, 1), jnp.float32),
                   jax.ShapeDtypeStruct((_B, _T, 1), jnp.float32),
                   jax.ShapeDtypeStruct((_B, 1, _MAXD), jnp.int32),
                   jax.ShapeDtypeStruct((_B, 1, 1), jnp.float32)],
    )(x, maskf, md, *W)


def _tc2_body(xr_ref, mlen_ref,
              pw10, pw11, pw12, pb1, pg1, pbb1, pw20, pw21, pw22, pb2, pg2, pbb2, plw, plb,
              ew10, ew11, ew12, eb1, eg1, ebb1, ew20, ew21, ew22, eb2, eg2, ebb2, elw, elb,
              pbins_ref, ebins_ref, ptab_ref, etab_ref,
              xr3_ref, mm_ref, pp_ref, ep_ref, pemb_ref, eemb_ref):
    pos = lax.broadcasted_iota(jnp.int32, (_MAXD, 1), 0).astype(jnp.float32)
    mlen = mlen_ref[0]
    maskc = pos >= mlen
    mm_ref[0] = maskc.astype(jnp.int32)
    xb = jnp.where(maskc, 0.0, xr_ref[0])
    PW = _load_w((pw10, pw11, pw12, pb1, pg1, pbb1, pw20, pw21, pw22, pb2, pg2, pbb2, plw, plb))
    pp = _pred_chain(xb, PW)
    pp = jnp.where(maskc, 0.0, pp)
    pp_ref[0] = pp
    pidx = jnp.sum((pbins_ref[...] < pp).astype(jnp.int32), axis=-1, keepdims=True)
    binid = lax.broadcasted_iota(jnp.int32, (1, _NBINS), 1)
    pemb_ref[0] = jnp.dot((binid == pidx).astype(jnp.float32), ptab_ref[...],
                          precision=_DEF)
    xr2 = xb + pp
    EW = _load_w((ew10, ew11, ew12, eb1, eg1, ebb1, ew20, ew21, ew22, eb2, eg2, ebb2, elw, elb))
    ep = _pred_chain(xr2, EW)
    ep = jnp.where(maskc, 0.0, ep)
    ep_ref[0] = ep
    eidx = jnp.sum((ebins_ref[...] < ep).astype(jnp.int32), axis=-1, keepdims=True)
    eemb_ref[0] = jnp.dot((binid == eidx).astype(jnp.float32), etab_ref[...],
                          precision=_DEF)
    xr3_ref[0] = xr2 + ep


def _run_tc2(xr, mlen, PW, EW, pbins, ebins, ptab, etab):
    wspec = [pl.BlockSpec(w.shape, lambda b: (0,) * w.ndim) for w in PW + EW]
    return pl.pallas_call(
        _tc2_body,
        grid=(_B,),
        in_specs=[pl.BlockSpec((1, _MAXD, _D), lambda b: (b, 0, 0)),
                  pl.BlockSpec((1, 1, 1), lambda b: (b, 0, 0))] + wspec +
                 [pl.BlockSpec((1, _NBINS), lambda b: (0, 0)),
                  pl.BlockSpec((1, _NBINS), lambda b: (0, 0)),
                  pl.BlockSpec((_NBINS, _D), lambda b: (0, 0)),
                  pl.BlockSpec((_NBINS, _D), lambda b: (0, 0))],
        out_specs=[pl.BlockSpec((1, _MAXD, _D), lambda b: (b, 0, 0)),
                   pl.BlockSpec((1, _MAXD, 1), lambda b: (b, 0, 0)),
                   pl.BlockSpec((1, _MAXD, 1), lambda b: (b, 0, 0)),
                   pl.BlockSpec((1, _MAXD, 1), lambda b: (b, 0, 0)),
                   pl.BlockSpec((1, _MAXD, _D), lambda b: (b, 0, 0)),
                   pl.BlockSpec((1, _MAXD, _D), lambda b: (b, 0, 0))],
        out_shape=[jax.ShapeDtypeStruct((_B, _MAXD, _D), jnp.float32),
                   jax.ShapeDtypeStruct((_B, _MAXD, 1), jnp.int32),
                   jax.ShapeDtypeStruct((_B, _MAXD, 1), jnp.float32),
                   jax.ShapeDtypeStruct((_B, _MAXD, 1), jnp.float32),
                   jax.ShapeDtypeStruct((_B, _MAXD, _D), jnp.float32),
                   jax.ShapeDtypeStruct((_B, _MAXD, _D), jnp.float32)],
    )(xr, mlen, *PW, *EW, pbins, ebins, ptab, etab)


def _sc_gather(table, idx):
    n, d = idx.shape[0], table.shape[1]
    nw = 32
    per_w = n // nw
    ch = 128
    n_ch = per_w // ch
    mesh = plsc.VectorSubcoreMesh(core_axis_name="c", subcore_axis_name="s")

    @functools.partial(
        pl.kernel, mesh=mesh,
        out_type=jax.ShapeDtypeStruct((n, d), jnp.float32),
        scratch_types=[pltpu.VMEM((per_w,), jnp.int32),
                       pltpu.VMEM((ch, d), jnp.float32),
                       pltpu.VMEM((ch, d), jnp.float32),
                       pltpu.SemaphoreType.DMA,
                       pltpu.SemaphoreType.DMA],
    )
    def k(table_hbm, idx_hbm, out_hbm, idx_v, buf0, buf1, sem0, sem1):
        wid = lax.axis_index("s") * 2 + lax.axis_index("c")
        base = wid * per_w
        pltpu.sync_copy(idx_hbm.at[pl.ds(base, per_w)], idx_v)
        bufs, sems = (buf0, buf1), (sem0, sem1)
        handles = [None] * n_ch
        handles[0] = pltpu.async_copy(
            table_hbm.at[idx_v.at[pl.ds(0, ch)]], bufs[0], sems[0])
        for ci in range(n_ch):
            if ci + 1 < n_ch:
                handles[ci + 1] = pltpu.async_copy(
                    table_hbm.at[idx_v.at[pl.ds((ci + 1) * ch, ch)]],
                    bufs[(ci + 1) % 2], sems[(ci + 1) % 2])
            handles[ci].wait()
            pltpu.sync_copy(bufs[ci % 2], out_hbm.at[pl.ds(base + ci * ch, ch)])

    return k(table, idx)


def _wpack(p):
    w1, w2 = p["conv1_w"], p["conv2_w"]
    return (jnp.asarray(w1[:, :, 0].T), jnp.asarray(w1[:, :, 1].T), jnp.asarray(w1[:, :, 2].T),
            p["conv1_b"].reshape(1, _D), p["ln1_g"].reshape(1, _D), p["ln1_b"].reshape(1, _D),
            jnp.asarray(w2[:, :, 0].T), jnp.asarray(w2[:, :, 1].T), jnp.asarray(w2[:, :, 2].T),
            p["conv2_b"].reshape(1, _D), p["ln2_g"].reshape(1, _D), p["ln2_b"].reshape(1, _D),
            p["lin_w"].reshape(_D, 1), p["lin_b"].reshape(1, 1))


def kernel(x, src_mask, max_dur, params):
    maskf = src_mask.astype(jnp.float32).reshape(_B, _T, 1)
    md = jnp.asarray(max_dur, jnp.float32).reshape(1, 1)
    DW = _wpack(params["dur"])
    PW = _wpack(params["pitch"])
    EW = _wpack(params["energy"])
    inf = jnp.full((1,), jnp.inf, jnp.float32)
    pbins = jnp.concatenate([params["pitch_bins"], inf]).reshape(1, _NBINS)
    ebins = jnp.concatenate([params["energy_bins"], inf]).reshape(1, _NBINS)

    ld3, dv3, idxg, mlen = _run_tc1(x, maskf, md, DW)
    xr = _sc_gather(x.reshape(_B * _T, _D), idxg.reshape(-1))
    xr = xr.reshape(_B, _MAXD, _D)
    xr3, mm, pp3, ep3, pitch_emb, energy_emb = _run_tc2(
        xr, mlen, PW, EW, pbins, ebins, params["pitch_emb"], params["energy_emb"])
    return (xr3, mm.reshape(_B, _MAXD).astype(bool),
            ld3.reshape(_B, _T), dv3.reshape(_B, _T),
            pp3.reshape(_B, _MAXD), pitch_emb,
            ep3.reshape(_B, _MAXD), energy_emb)
